# baseline (speedup 1.0000x reference)
import functools

import jax
import jax.numpy as jnp
from jax import lax
from jax.experimental import pallas as pl
from jax.experimental.pallas import tpu as pltpu
from jax.experimental.pallas import tpu_sc as plsc


@functools.lru_cache(maxsize=None)
def _make_gather(vocab: int, embed: int, batch: int):
    info = plsc.get_sparse_core_info()
    nw = info.num_cores * info.num_subcores
    b_per_w = batch // nw

    chunk = 1024
    block = 1024
    k = block // chunk
    n_blocks = b_per_w // block
    assert b_per_w % block == 0 and block % chunk == 0

    mesh = plsc.VectorSubcoreMesh(core_axis_name="c", subcore_axis_name="s")

    @functools.partial(
        pl.kernel,
        mesh=mesh,
        out_type=jax.ShapeDtypeStruct((batch, embed), jnp.float32),
        scratch_types=[
            pltpu.VMEM((block,), jnp.int32),
            pltpu.VMEM((block, embed), jnp.float32),
            pltpu.SemaphoreType.DMA,
        ],
        compiler_params=pltpu.CompilerParams(use_tc_tiling_on_sc=False),
    )
    def gather_kernel(table_hbm, idx_hbm, out_hbm, idx_v, rows_v, sem):
        wid = lax.axis_index("s") * info.num_cores + lax.axis_index("c")
        base = wid * b_per_w

        def body(blk, carry):
            off = base + blk * block
            pltpu.sync_copy(idx_hbm.at[pl.ds(off, block)], idx_v)
            copies = [
                pltpu.async_copy(
                    table_hbm.at[idx_v.at[pl.ds(j * chunk, chunk)]],
                    rows_v.at[pl.ds(j * chunk, chunk)],
                    sem,
                )
                for j in range(k)
            ]
            for c in copies:
                c.wait()
            pltpu.sync_copy(rows_v, out_hbm.at[pl.ds(off, block)])
            return carry

        lax.fori_loop(0, n_blocks, body, 0)

    return gather_kernel


def kernel(indices, table):
    batch, seq = indices.shape
    vocab, embed = table.shape
    idx_flat = indices.reshape(-1).astype(jnp.int32)
    out = _make_gather(vocab, embed, batch * seq)(table, idx_flat)
    return out.reshape(batch, seq, embed)


# speedup vs baseline: 1.0167x; 1.0167x over previous
import functools

import jax
import jax.numpy as jnp
from jax import lax
from jax.experimental import pallas as pl
from jax.experimental.pallas import tpu as pltpu
from jax.experimental.pallas import tpu_sc as plsc


@functools.lru_cache(maxsize=None)
def _make_gather(vocab: int, embed: int, batch: int):
    info = plsc.get_sparse_core_info()
    nw = info.num_cores * info.num_subcores
    b_per_w = batch // nw

    nbuf = 4
    block = 400
    n_blocks = b_per_w // block
    outer = n_blocks // nbuf
    assert b_per_w % block == 0 and n_blocks % nbuf == 0 and outer >= 2

    mesh = plsc.VectorSubcoreMesh(core_axis_name="c", subcore_axis_name="s")

    @functools.partial(
        pl.kernel,
        mesh=mesh,
        out_type=jax.ShapeDtypeStruct((batch, embed), jnp.float32),
        scratch_types=[
            pltpu.VMEM((nbuf, block), jnp.int32),
            pltpu.VMEM((nbuf, block, embed), jnp.float32),
            pltpu.SemaphoreType.DMA((nbuf,)),
            pltpu.SemaphoreType.DMA((nbuf,)),
            pltpu.SemaphoreType.DMA((nbuf,)),
        ],
        compiler_params=pltpu.CompilerParams(use_tc_tiling_on_sc=False),
    )
    def gather_kernel(table_hbm, idx_hbm, out_hbm, idx_v, rows_v, gsem, wsem, isem):
        wid = lax.axis_index("s") * info.num_cores + lax.axis_index("c")
        base = wid * b_per_w

        def idx_copy(g, b):
            return pltpu.make_async_copy(
                idx_hbm.at[pl.ds(base + g * block, block)], idx_v.at[b], isem.at[b])

        def gather_copy(b):
            return pltpu.make_async_copy(
                table_hbm.at[idx_v.at[b]], rows_v.at[b], gsem.at[b])

        def wb_copy(g, b):
            return pltpu.make_async_copy(
                rows_v.at[b], out_hbm.at[pl.ds(base + g * block, block)], wsem.at[b])

        def step(g, b, wait_wb, drain_prev, prefetch):
            p = (b - 1) % nbuf
            idx_copy(g, b).wait()
            if wait_wb:
                wb_copy(g - nbuf, b).wait()
            gather_copy(b).start()
            if drain_prev:
                gather_copy(p).wait()
                wb_copy(g - 1, p).start()
            if prefetch:
                idx_copy(g - 1 + nbuf, p).start()

        for b in range(nbuf):
            idx_copy(b, b).start()

        for g in range(nbuf):
            step(g, g, wait_wb=False, drain_prev=g >= 1, prefetch=g >= 1)

        def outer_body(o, carry):
            g0 = o * nbuf
            for j in range(nbuf):
                step(g0 + j, j, wait_wb=True, drain_prev=True, prefetch=True)
            return carry

        lax.fori_loop(1, outer - 1, outer_body, 0)

        for i, g in enumerate(range(n_blocks - nbuf, n_blocks)):
            step(g, i, wait_wb=True, drain_prev=True,
                 prefetch=g <= n_blocks - nbuf)

        b_last = (n_blocks - 1) % nbuf
        gather_copy(b_last).wait()
        wb_copy(n_blocks - 1, b_last).start()
        for b in range(nbuf):
            wb_copy(n_blocks - nbuf + b, b).wait()

    return gather_kernel


def kernel(indices, table):
    batch, seq = indices.shape
    vocab, embed = table.shape
    idx_flat = indices.reshape(-1).astype(jnp.int32)
    out = _make_gather(vocab, embed, batch * seq)(table, idx_flat)
    return out.reshape(batch, seq, embed)
